# baseline (speedup 1.0000x reference)
import jax
import jax.numpy as jnp
from jax import lax
from jax.experimental import pallas as pl
from jax.experimental.pallas import tpu as pltpu
from jax.experimental.pallas import tpu_sc as plsc

_L = 50
_PE = 1000
_NW = 32
_BTW = 128 // _NW
_NSLAB = _L * _BTW
_NBUF = 3


def _sc_body(x_hbm, idx_hbm, pe_hbm, out_hbm,
             pebuf, xbuf, ibuf,
             xs0, xs1, xs2, is0, is1, is2, ss0, ss1, ss2):
    xsems = (xs0, xs1, xs2)
    isems = (is0, is1, is2)
    ssems = (ss0, ss1, ss2)
    wid = lax.axis_index("s") * 2 + lax.axis_index("c")
    bt0 = wid * _BTW

    def slab_lbt(k):
        return k >> 2, bt0 + (k & 3)

    def load_cp(k, s):
        l, bt = slab_lbt(k)
        return (
            pltpu.make_async_copy(
                x_hbm.at[l, :, pl.ds(bt, 1)], xbuf.at[s], xsems[s]),
            pltpu.make_async_copy(
                idx_hbm.at[l, bt], ibuf.at[s], isems[s]),
        )

    def store_cp(k, s):
        l, bt = slab_lbt(k)
        return pltpu.make_async_copy(
            xbuf.at[s], out_hbm.at[l, :, pl.ds(bt, 1)], ssems[s])

    def start_load(k, s):
        xcp, icp = load_cp(k, s)
        xcp.start()
        icp.start()

    def compute(s):
        @plsc.parallel_loop(0, 128, step=16, unroll=2)
        def g_body(g):
            iv = ibuf[s, pl.ds(g, 16)]
            base = iv * 65
            for dt in range(0, 8, 2):
                vals = [plsc.load_gather(pebuf, [base + (dt * 8 + d)])
                        for d in range(16)]
                for d in range(16):
                    plsc.addupdate(
                        xbuf.at[s, dt + (d >> 3), 0, d & 7, pl.ds(g, 16)],
                        vals[d])

    def slab(k, s):
        @pl.when(k < _NSLAB)
        def _():
            xcp, icp = load_cp(k, s)
            xcp.wait()
            icp.wait()
            compute(s)
            store_cp(k, s).start()

            @pl.when(k >= 1)
            def _():
                store_cp(k - 1, (s - 1) % _NBUF).wait()

            @pl.when(k + 2 < _NSLAB)
            def _():
                start_load(k + 2, (s + 2) % _NBUF)

    start_load(0, 0)
    start_load(1, 1)
    pltpu.sync_copy(pe_hbm, pebuf)

    ngrp = -(-_NSLAB // _NBUF)

    def group(g, c):
        for s in range(_NBUF):
            slab(g * _NBUF + s, s)
        return c

    lax.fori_loop(0, ngrp, group, 0)
    store_cp(_NSLAB - 1, (_NSLAB - 1) % _NBUF).wait()


def kernel(X, index, pe):
    x6 = jnp.transpose(
        jnp.transpose(X, (1, 2, 0)).reshape(_L, 8, 8, 128, 128),
        (0, 1, 3, 2, 4))
    i6 = jnp.transpose(index, (1, 2, 0)).reshape(_L, 128, 128)
    pe1 = jnp.pad(pe, ((0, 0), (0, 1))).reshape(_PE * 65)

    mesh = plsc.VectorSubcoreMesh(core_axis_name="c", subcore_axis_name="s")
    out6 = pl.kernel(
        _sc_body,
        out_type=jax.ShapeDtypeStruct((_L, 8, 128, 8, 128), jnp.float32),
        mesh=mesh,
        compiler_params=pltpu.CompilerParams(
            use_tc_tiling_on_sc=False, needs_layout_passes=False),
        scratch_types=[
            pltpu.VMEM((_PE * 65,), jnp.float32),
            pltpu.VMEM((_NBUF, 8, 1, 8, 128), jnp.float32),
            pltpu.VMEM((_NBUF, 128), jnp.int32),
            pltpu.SemaphoreType.DMA,
            pltpu.SemaphoreType.DMA,
            pltpu.SemaphoreType.DMA,
            pltpu.SemaphoreType.DMA,
            pltpu.SemaphoreType.DMA,
            pltpu.SemaphoreType.DMA,
            pltpu.SemaphoreType.DMA,
            pltpu.SemaphoreType.DMA,
            pltpu.SemaphoreType.DMA,
        ],
    )(x6, i6, pe1)

    out = jnp.transpose(
        jnp.transpose(out6, (0, 1, 3, 2, 4)).reshape(_L, 64, 16384),
        (2, 0, 1))
    return out


# speedup vs baseline: 1.2782x; 1.2782x over previous
import jax
import jax.numpy as jnp
from jax import lax
from jax.experimental import pallas as pl
from jax.experimental.pallas import tpu as pltpu
from jax.experimental.pallas import tpu_sc as plsc

_L = 50
_PE = 1000
_NW = 32
_BTW = 128 // _NW
_NSLAB = _L * _BTW
_NBUF = 3


def _sc_body(x_hbm, idx_hbm, pe_hbm, out_hbm,
             pebuf, xbuf, ibuf,
             xs0, xs1, xs2, is0, is1, is2, ss0, ss1, ss2):
    xsems = (xs0, xs1, xs2)
    isems = (is0, is1, is2)
    ssems = (ss0, ss1, ss2)
    wid = lax.axis_index("s") * 2 + lax.axis_index("c")
    bt0 = wid * _BTW

    def slab_lbt(k):
        return k >> 2, bt0 + (k & 3)

    def load_cp(k, s):
        l, bt = slab_lbt(k)
        return (
            pltpu.make_async_copy(
                x_hbm.at[l, :, pl.ds(bt, 1)], xbuf.at[s], xsems[s]),
            pltpu.make_async_copy(
                idx_hbm.at[l, bt], ibuf.at[s], isems[s]),
        )

    def store_cp(k, s):
        l, bt = slab_lbt(k)
        return pltpu.make_async_copy(
            xbuf.at[s], out_hbm.at[l, :, pl.ds(bt, 1)], ssems[s])

    def start_load(k, s):
        xcp, icp = load_cp(k, s)
        xcp.start()
        icp.start()

    def compute(s):
        @plsc.parallel_loop(0, 128, step=16)
        def g_body(g):
            iv = ibuf[s, pl.ds(g, 16)]
            base = iv * 65
            nsl = _PE * 65 - 64
            bvals = [base + di for di in range(8)]
            for dt in range(8):
                vals = [
                    plsc.load_gather(
                        pebuf.at[pl.ds(dt * 8, nsl)], [bvals[di]])
                    for di in range(8)]
                for di in range(8):
                    plsc.addupdate(xbuf.at[s, dt, 0, di, pl.ds(g, 16)],
                                   vals[di])

    def slab(k, s):
        @pl.when(k < _NSLAB)
        def _():
            xcp, icp = load_cp(k, s)
            xcp.wait()
            icp.wait()
            compute(s)
            store_cp(k, s).start()

            @pl.when(k >= 1)
            def _():
                store_cp(k - 1, (s - 1) % _NBUF).wait()

            @pl.when(k + 2 < _NSLAB)
            def _():
                start_load(k + 2, (s + 2) % _NBUF)

    start_load(0, 0)
    start_load(1, 1)
    pltpu.sync_copy(pe_hbm, pebuf)

    ngrp = -(-_NSLAB // _NBUF)

    def group(g, c):
        for s in range(_NBUF):
            slab(g * _NBUF + s, s)
        return c

    lax.fori_loop(0, ngrp, group, 0)
    store_cp(_NSLAB - 1, (_NSLAB - 1) % _NBUF).wait()


def kernel(X, index, pe):
    x6 = jnp.transpose(
        jnp.transpose(X, (1, 2, 0)).reshape(_L, 8, 8, 128, 128),
        (0, 1, 3, 2, 4))
    i6 = jnp.transpose(index, (1, 2, 0)).reshape(_L, 128, 128)
    pe1 = jnp.pad(pe, ((0, 0), (0, 1))).reshape(_PE * 65)

    mesh = plsc.VectorSubcoreMesh(core_axis_name="c", subcore_axis_name="s")
    out6 = pl.kernel(
        _sc_body,
        out_type=jax.ShapeDtypeStruct((_L, 8, 128, 8, 128), jnp.float32),
        mesh=mesh,
        compiler_params=pltpu.CompilerParams(
            use_tc_tiling_on_sc=False, needs_layout_passes=False),
        scratch_types=[
            pltpu.VMEM((_PE * 65,), jnp.float32),
            pltpu.VMEM((_NBUF, 8, 1, 8, 128), jnp.float32),
            pltpu.VMEM((_NBUF, 128), jnp.int32),
            pltpu.SemaphoreType.DMA,
            pltpu.SemaphoreType.DMA,
            pltpu.SemaphoreType.DMA,
            pltpu.SemaphoreType.DMA,
            pltpu.SemaphoreType.DMA,
            pltpu.SemaphoreType.DMA,
            pltpu.SemaphoreType.DMA,
            pltpu.SemaphoreType.DMA,
            pltpu.SemaphoreType.DMA,
        ],
    )(x6, i6, pe1)

    out = jnp.transpose(
        jnp.transpose(out6, (0, 1, 3, 2, 4)).reshape(_L, 64, 16384),
        (2, 0, 1))
    return out


# speedup vs baseline: 1.5234x; 1.1918x over previous
import jax
import jax.numpy as jnp
from jax import lax
from jax.experimental import pallas as pl
from jax.experimental.pallas import tpu as pltpu
from jax.experimental.pallas import tpu_sc as plsc

_L = 50
_PE = 1000
_NW = 32
_BTW = 128 // _NW
_NSLAB = _L * _BTW
_NBUF = 4


def _sc_body(x_hbm, idx_hbm, pe_hbm, out_hbm,
             pebuf, xbuf, ibuf,
             xs0, xs1, xs2, xs3, is0, is1, is2, is3,
             ss0, ss1, ss2, ss3):
    xsems = (xs0, xs1, xs2, xs3)
    isems = (is0, is1, is2, is3)
    ssems = (ss0, ss1, ss2, ss3)
    wid = lax.axis_index("s") * 2 + lax.axis_index("c")
    bt0 = wid * _BTW

    def slab_lbt(k):
        return k >> 2, bt0 + (k & 3)

    def load_cp(k, s):
        l, bt = slab_lbt(k)
        return (
            pltpu.make_async_copy(
                x_hbm.at[l, :, pl.ds(bt, 1)], xbuf.at[s], xsems[s]),
            pltpu.make_async_copy(
                idx_hbm.at[l, bt], ibuf.at[s], isems[s]),
        )

    def store_cp(k, s):
        l, bt = slab_lbt(k)
        return pltpu.make_async_copy(
            xbuf.at[s], out_hbm.at[l, :, pl.ds(bt, 1)], ssems[s])

    def start_load(k, s):
        xcp, icp = load_cp(k, s)
        xcp.start()
        icp.start()

    def compute(s):
        @plsc.parallel_loop(0, 128, step=16)
        def g_body(g):
            iv = ibuf[s, pl.ds(g, 16)]
            base = iv * 65
            nsl = _PE * 65 - 64
            bvals = [base + di for di in range(8)]
            for dt in range(8):
                vals = [
                    plsc.load_gather(
                        pebuf.at[pl.ds(dt * 8, nsl)], [bvals[di]])
                    for di in range(8)]
                for di in range(8):
                    plsc.addupdate(xbuf.at[s, dt, 0, di, pl.ds(g, 16)],
                                   vals[di])

    def slab(k, s):
        @pl.when(k < _NSLAB)
        def _():
            xcp, icp = load_cp(k, s)
            xcp.wait()
            icp.wait()

            @pl.when(k >= 2)
            def _():
                store_cp(k - 2, (s + 2) % _NBUF).wait()

            @pl.when(k + 2 < _NSLAB)
            def _():
                start_load(k + 2, (s + 2) % _NBUF)

            compute(s)
            store_cp(k, s).start()

    start_load(0, 0)
    start_load(1, 1)
    pltpu.sync_copy(pe_hbm, pebuf)

    ngrp = -(-_NSLAB // _NBUF)

    def group(g, c):
        for s in range(_NBUF):
            slab(g * _NBUF + s, s)
        return c

    lax.fori_loop(0, ngrp, group, 0)
    store_cp(_NSLAB - 2, (_NSLAB - 2) % _NBUF).wait()
    store_cp(_NSLAB - 1, (_NSLAB - 1) % _NBUF).wait()


def kernel(X, index, pe):
    x6 = jnp.transpose(
        jnp.transpose(X, (1, 2, 0)).reshape(_L, 8, 8, 128, 128),
        (0, 1, 3, 2, 4))
    i6 = jnp.transpose(index, (1, 2, 0)).reshape(_L, 128, 128)
    pe1 = jnp.pad(pe, ((0, 0), (0, 1))).reshape(_PE * 65)

    mesh = plsc.VectorSubcoreMesh(core_axis_name="c", subcore_axis_name="s")
    out6 = pl.kernel(
        _sc_body,
        out_type=jax.ShapeDtypeStruct((_L, 8, 128, 8, 128), jnp.float32),
        mesh=mesh,
        compiler_params=pltpu.CompilerParams(
            use_tc_tiling_on_sc=False, needs_layout_passes=False),
        scratch_types=[
            pltpu.VMEM((_PE * 65,), jnp.float32),
            pltpu.VMEM((_NBUF, 8, 1, 8, 128), jnp.float32),
            pltpu.VMEM((_NBUF, 128), jnp.int32),
        ] + [pltpu.SemaphoreType.DMA] * (3 * _NBUF),
    )(x6, i6, pe1)

    out = jnp.transpose(
        jnp.transpose(out6, (0, 1, 3, 2, 4)).reshape(_L, 64, 16384),
        (2, 0, 1))
    return out
